# baseline (speedup 1.0000x reference)
import functools

import jax
import jax.numpy as jnp
from jax import lax
from jax.experimental import pallas as pl
from jax.experimental.pallas import tpu as pltpu
from jax.experimental.pallas import tpu_sc as plsc

BATCH_N = 16384
DIM_N = 16
CHUNK = 128


def _make_sc_kernel(B, D):
    info = plsc.get_sparse_core_info()
    NC, NS = info.num_cores, info.num_subcores
    NW = NC * NS
    b_per_w = B // NW
    n_chunks = b_per_w // CHUNK
    mesh = plsc.VectorSubcoreMesh(core_axis_name="c", subcore_axis_name="s")

    @functools.partial(
        pl.kernel,
        mesh=mesh,
        out_type=jax.ShapeDtypeStruct((B, D), jnp.float32),
        compiler_params=pltpu.CompilerParams(use_tc_tiling_on_sc=False),
        scratch_types=[
            pltpu.VMEM((n_chunks, CHUNK), jnp.int32),
            pltpu.VMEM((n_chunks, CHUNK), jnp.int32),
            pltpu.VMEM((b_per_w, D), jnp.float32),
            pltpu.VMEM((b_per_w, D), jnp.float32),
            pltpu.SemaphoreType.DMA,
            pltpu.SemaphoreType.DMA,
        ],
    )
    def k(user_hbm, item_hbm, uemb_hbm, iemb_hbm, out_hbm,
          uidx_v, iidx_v, urows_v, irows_v, sem_u, sem_i):
        wid = lax.axis_index("s") * NC + lax.axis_index("c")
        base = wid * n_chunks
        pltpu.sync_copy(user_hbm.at[pl.ds(base, n_chunks)], uidx_v)
        pltpu.sync_copy(item_hbm.at[pl.ds(base, n_chunks)], iidx_v)
        copies = []
        for j in range(n_chunks):
            copies.append(pltpu.async_copy(
                uemb_hbm.at[uidx_v.at[j]],
                urows_v.at[pl.ds(j * CHUNK, CHUNK)], sem_u))
            copies.append(pltpu.async_copy(
                iemb_hbm.at[iidx_v.at[j]],
                irows_v.at[pl.ds(j * CHUNK, CHUNK)], sem_i))
        for c in copies:
            c.wait()

        def body(i, carry):
            urows_v[i, :] = urows_v[i, :] * irows_v[i, :]
            return carry

        lax.fori_loop(0, b_per_w, body, 0, unroll=8)
        pltpu.sync_copy(urows_v, out_hbm.at[pl.ds(wid * b_per_w, b_per_w)])

    return k


def kernel(user, item, user_emb, item_emb):
    B, D = BATCH_N, DIM_N
    k = _make_sc_kernel(B, D)
    user2 = user.reshape(B // CHUNK, CHUNK)
    item2 = item.reshape(B // CHUNK, CHUNK)
    return k(user2, item2, user_emb, item_emb)
